# Initial kernel scaffold; baseline (speedup 1.0000x reference)
import functools

import jax
import jax.numpy as jnp
from jax import lax
from jax.experimental import pallas as pl
from jax.experimental.pallas import tpu as pltpu
from jax.experimental.pallas import tpu_sc as plsc

B = 16384
K = 200
NC = 2
NS = 16
L = 16
NW = NC * NS
PER_W = B // NW
C = 32
NCHUNK = PER_W // C
GROUPS = C // L


def _sc_body(head_hbm, rel_hbm, tail_hbm, eh_hbm, et_hbm, r_hbm, ri_hbm,
             out_hbm, head_v, rel_v, tail_v, out_v, bufs, sems):
    wid = lax.axis_index("s") * NC + lax.axis_index("c")
    base = wid * PER_W

    pltpu.sync_copy(head_hbm.at[pl.ds(base, PER_W)], head_v)
    pltpu.sync_copy(rel_hbm.at[pl.ds(base, PER_W)], rel_v)
    pltpu.sync_copy(tail_hbm.at[pl.ds(base, PER_W)], tail_v)

    def start(c, slot):
        hi = head_v.at[pl.ds(c * C, C)]
        re = rel_v.at[pl.ds(c * C, C)]
        ti = tail_v.at[pl.ds(c * C, C)]
        hh, ht, rr, rri, th, tt = bufs[slot]
        sem = sems[slot]
        return [
            pltpu.async_copy(eh_hbm.at[hi], hh, sem),
            pltpu.async_copy(et_hbm.at[hi], ht, sem),
            pltpu.async_copy(r_hbm.at[re], rr, sem),
            pltpu.async_copy(ri_hbm.at[re], rri, sem),
            pltpu.async_copy(eh_hbm.at[ti], th, sem),
            pltpu.async_copy(et_hbm.at[ti], tt, sem),
        ]

    lane = lax.iota(jnp.int32, L)
    zero = jnp.zeros((L,), jnp.float32)

    def compute(c, slot):
        hh, ht, rr, rri, th, tt = bufs[slot]
        for g in range(GROUPS):
            rows = lane + (g * L)

            def kbody(k, carry):
                a1, a2 = carry
                cols = jnp.zeros((L,), jnp.int32) + k
                hhv = plsc.load_gather(hh, [rows, cols])
                htv = plsc.load_gather(ht, [rows, cols])
                rv = plsc.load_gather(rr, [rows, cols])
                riv = plsc.load_gather(rri, [rows, cols])
                thv = plsc.load_gather(th, [rows, cols])
                ttv = plsc.load_gather(tt, [rows, cols])
                return a1 + hhv * rv * ttv, a2 + thv * riv * htv

            a1, a2 = lax.fori_loop(0, K, kbody, (zero, zero), unroll=4)
            score = jnp.clip((a1 + a2) * 0.5, -20.0, 20.0)
            out_v[pl.ds(c * C + g * L, L)] = score

    cps = start(0, 0)
    for c in range(NCHUNK):
        nxt = None
        if c + 1 < NCHUNK:
            nxt = start(c + 1, (c + 1) % 2)
        for cp in cps:
            cp.wait()
        compute(c, c % 2)
        cps = nxt

    pltpu.sync_copy(out_v, out_hbm.at[pl.ds(base, PER_W)])


def _build():
    mesh = plsc.VectorSubcoreMesh(
        core_axis_name="c", subcore_axis_name="s", num_cores=NC,
        num_subcores=NS)
    row_buf = lambda: pltpu.VMEM((C, K), jnp.float32)
    scratch = [
        pltpu.VMEM((PER_W,), jnp.int32),
        pltpu.VMEM((PER_W,), jnp.int32),
        pltpu.VMEM((PER_W,), jnp.int32),
        pltpu.VMEM((PER_W,), jnp.float32),
        [[row_buf() for _ in range(6)] for _ in range(2)],
        [pltpu.SemaphoreType.DMA, pltpu.SemaphoreType.DMA],
    ]
    return pl.kernel(
        _sc_body,
        out_type=jax.ShapeDtypeStruct((B,), jnp.float32),
        mesh=mesh,
        scratch_types=scratch,
    )


_sc_kernel = _build()


@jax.jit
def kernel(head, rel, tail, embed_eh, embed_et, embed_r, embed_ri):
    head = head.astype(jnp.int32)
    rel = rel.astype(jnp.int32)
    tail = tail.astype(jnp.int32)
    return _sc_kernel(head, rel, tail, embed_eh, embed_et, embed_r,
                      embed_ri)


# speedup vs baseline: 1.3559x; 1.3559x over previous
import functools

import jax
import jax.numpy as jnp
from jax import lax
from jax.experimental import pallas as pl
from jax.experimental.pallas import tpu as pltpu
from jax.experimental.pallas import tpu_sc as plsc

B = 16384
K = 200
NC = 2
NS = 16
L = 16
NW = NC * NS
PER_W = B // NW
C = 32
NCHUNK = PER_W // C
GROUPS = C // L
NSLOT = 3


def _sc_body(ent_hbm, rel_hbm, eh_hbm, et_hbm, r_hbm, ri_hbm,
             out_hbm, ent_v, rel_v, out_v, bufs, sems):
    wid = lax.axis_index("s") * NC + lax.axis_index("c")
    base = wid * PER_W

    pltpu.sync_copy(ent_hbm.at[pl.ds(base * 2, 2 * PER_W)], ent_v)
    pltpu.sync_copy(rel_hbm.at[pl.ds(base, PER_W)], rel_v)

    def start(c):
        slot = c % NSLOT
        ei = ent_v.at[pl.ds(c * 2 * C, 2 * C)]
        re = rel_v.at[pl.ds(c * C, C)]
        eh_b, et_b, r_b, ri_b = bufs[slot]
        sem = sems[slot]
        return [
            pltpu.async_copy(eh_hbm.at[ei], eh_b, sem),
            pltpu.async_copy(et_hbm.at[ei], et_b, sem),
            pltpu.async_copy(r_hbm.at[re], r_b, sem),
            pltpu.async_copy(ri_hbm.at[re], ri_b, sem),
        ]

    lane = lax.iota(jnp.int32, L)
    zero = jnp.zeros((L,), jnp.float32)

    def compute(c):
        slot = c % NSLOT
        eh_b, et_b, r_b, ri_b = bufs[slot]
        for g in range(GROUPS):
            rows = lane + (g * L)
            rows_t = rows + C

            def kbody(k, carry):
                a1, a2 = carry
                cols = jnp.full((L,), 0, jnp.int32) + k
                hh = plsc.load_gather(eh_b, [rows, cols])
                th = plsc.load_gather(eh_b, [rows_t, cols])
                ht = plsc.load_gather(et_b, [rows, cols])
                tt = plsc.load_gather(et_b, [rows_t, cols])
                rv = plsc.load_gather(r_b, [rows, cols])
                riv = plsc.load_gather(ri_b, [rows, cols])
                return a1 + hh * rv * tt, a2 + th * riv * ht

            a1, a2 = lax.fori_loop(0, K, kbody, (zero, zero), unroll=4)
            score = jnp.clip((a1 + a2) * 0.5, -20.0, 20.0)
            out_v[pl.ds(c * C + g * L, L)] = score

    cps = {}
    for c in range(min(NSLOT, NCHUNK)):
        cps[c] = start(c)
    for c in range(NCHUNK):
        for cp in cps.pop(c):
            cp.wait()
        compute(c)
        if c + NSLOT < NCHUNK:
            cps[c + NSLOT] = start(c + NSLOT)

    pltpu.sync_copy(out_v, out_hbm.at[pl.ds(base, PER_W)])


@functools.cache
def _build():
    mesh = plsc.VectorSubcoreMesh(
        core_axis_name="c", subcore_axis_name="s", num_cores=NC,
        num_subcores=NS)
    slot = lambda: [
        pltpu.VMEM((2 * C, K), jnp.float32),
        pltpu.VMEM((2 * C, K), jnp.float32),
        pltpu.VMEM((C, K), jnp.float32),
        pltpu.VMEM((C, K), jnp.float32),
    ]
    scratch = [
        pltpu.VMEM((2 * PER_W,), jnp.int32),
        pltpu.VMEM((PER_W,), jnp.int32),
        pltpu.VMEM((PER_W,), jnp.float32),
        [slot() for _ in range(NSLOT)],
        [pltpu.SemaphoreType.DMA for _ in range(NSLOT)],
    ]
    return pl.kernel(
        _sc_body,
        out_type=jax.ShapeDtypeStruct((B,), jnp.float32),
        mesh=mesh,
        scratch_types=scratch,
        compiler_params=pltpu.CompilerParams(
            use_tc_tiling_on_sc=False, needs_layout_passes=False),
    )


@jax.jit
def kernel(head, rel, tail, embed_eh, embed_et, embed_r, embed_ri):
    head = head.astype(jnp.int32)
    rel = rel.astype(jnp.int32)
    tail = tail.astype(jnp.int32)
    ent = jnp.stack(
        [head.reshape(NW, NCHUNK, C), tail.reshape(NW, NCHUNK, C)],
        axis=2).reshape(2 * B)
    return _build()(ent, rel, embed_eh, embed_et, embed_r, embed_ri)


# speedup vs baseline: 1.3623x; 1.0047x over previous
import functools

import jax
import jax.numpy as jnp
from jax import lax
from jax.experimental import pallas as pl
from jax.experimental.pallas import tpu as pltpu
from jax.experimental.pallas import tpu_sc as plsc

B = 16384
K = 200
NC = 2
NS = 16
L = 16
NW = NC * NS
PER_W = B // NW
C = 16
NCHUNK = PER_W // C
GROUPS = C // L
NSLOT = 6


def _sc_body(ent_hbm, rel_hbm, eh_hbm, et_hbm, r_hbm, ri_hbm,
             out_hbm, ent_v, rel_v, out_v, bufs, sems):
    wid = lax.axis_index("s") * NC + lax.axis_index("c")
    base = wid * PER_W

    pltpu.sync_copy(ent_hbm.at[pl.ds(base * 2, 2 * PER_W)], ent_v)
    pltpu.sync_copy(rel_hbm.at[pl.ds(base, PER_W)], rel_v)

    def start(c):
        slot = c % NSLOT
        ei = ent_v.at[pl.ds(c * 2 * C, 2 * C)]
        re = rel_v.at[pl.ds(c * C, C)]
        eh_b, et_b, r_b, ri_b = bufs[slot]
        sem = sems[slot]
        return [
            pltpu.async_copy(eh_hbm.at[ei], eh_b, sem),
            pltpu.async_copy(et_hbm.at[ei], et_b, sem),
            pltpu.async_copy(r_hbm.at[re], r_b, sem),
            pltpu.async_copy(ri_hbm.at[re], ri_b, sem),
        ]

    lane = lax.iota(jnp.int32, L)
    zero = jnp.zeros((L,), jnp.float32)

    def compute(c):
        slot = c % NSLOT
        eh_b, et_b, r_b, ri_b = bufs[slot]
        for g in range(GROUPS):
            rows = lane + (g * L)
            rows_t = rows + C

            def kbody(k, carry):
                a1, a2 = carry
                cols = jnp.full((L,), 0, jnp.int32) + k
                hh = plsc.load_gather(eh_b, [rows, cols])
                th = plsc.load_gather(eh_b, [rows_t, cols])
                ht = plsc.load_gather(et_b, [rows, cols])
                tt = plsc.load_gather(et_b, [rows_t, cols])
                rv = plsc.load_gather(r_b, [rows, cols])
                riv = plsc.load_gather(ri_b, [rows, cols])
                return a1 + hh * rv * tt, a2 + th * riv * ht

            a1, a2 = lax.fori_loop(0, K, kbody, (zero, zero), unroll=4)
            score = jnp.clip((a1 + a2) * 0.5, -20.0, 20.0)
            out_v[pl.ds(c * C + g * L, L)] = score

    cps = {}
    for c in range(min(NSLOT, NCHUNK)):
        cps[c] = start(c)
    for c in range(NCHUNK):
        for cp in cps.pop(c):
            cp.wait()
        compute(c)
        if c + NSLOT < NCHUNK:
            cps[c + NSLOT] = start(c + NSLOT)

    pltpu.sync_copy(out_v, out_hbm.at[pl.ds(base, PER_W)])


@functools.cache
def _build():
    mesh = plsc.VectorSubcoreMesh(
        core_axis_name="c", subcore_axis_name="s", num_cores=NC,
        num_subcores=NS)
    slot = lambda: [
        pltpu.VMEM((2 * C, K), jnp.float32),
        pltpu.VMEM((2 * C, K), jnp.float32),
        pltpu.VMEM((C, K), jnp.float32),
        pltpu.VMEM((C, K), jnp.float32),
    ]
    scratch = [
        pltpu.VMEM((2 * PER_W,), jnp.int32),
        pltpu.VMEM((PER_W,), jnp.int32),
        pltpu.VMEM((PER_W,), jnp.float32),
        [slot() for _ in range(NSLOT)],
        [pltpu.SemaphoreType.DMA for _ in range(NSLOT)],
    ]
    return pl.kernel(
        _sc_body,
        out_type=jax.ShapeDtypeStruct((B,), jnp.float32),
        mesh=mesh,
        scratch_types=scratch,
        compiler_params=pltpu.CompilerParams(
            use_tc_tiling_on_sc=False, needs_layout_passes=False),
    )


@jax.jit
def kernel(head, rel, tail, embed_eh, embed_et, embed_r, embed_ri):
    head = head.astype(jnp.int32)
    rel = rel.astype(jnp.int32)
    tail = tail.astype(jnp.int32)
    ent = jnp.stack(
        [head.reshape(NW, NCHUNK, C), tail.reshape(NW, NCHUNK, C)],
        axis=2).reshape(2 * B)
    return _build()(ent, rel, embed_eh, embed_et, embed_r, embed_ri)
